# baseline (speedup 1.0000x reference)
import functools

import jax
import jax.numpy as jnp
from jax.experimental import pallas as pl
from jax.experimental.pallas import tpu as pltpu


def _se_step(x_ref, w1t_ref, w2t_ref, o_ref, *, inv_hw):
    slabs = x_ref[...]
    avg = jnp.sum(slabs, axis=1) * inv_hw
    h = jnp.maximum(
        jnp.dot(avg, w1t_ref[...], preferred_element_type=jnp.float32), 0.0)
    gate = jax.nn.sigmoid(
        jnp.dot(h, w2t_ref[...], preferred_element_type=jnp.float32))
    o_ref[...] = slabs * gate[:, None, :]


def kernel(x_nchw, w1, w2):
    B, C, H, W = x_nchw.shape
    HW = H * W
    Cr = w1.shape[0]

    x_flat = jnp.transpose(x_nchw, (0, 2, 3, 1)).reshape(B, HW, C)
    w1t = w1.T
    w2t = w2.T

    out_flat = pl.pallas_call(
        functools.partial(_se_step, inv_hw=1.0 / float(HW)),
        out_shape=jax.ShapeDtypeStruct((B, HW, C), x_nchw.dtype),
        grid=(B // 4,),
        in_specs=[
            pl.BlockSpec((4, HW, C), lambda b: (b, 0, 0)),
            pl.BlockSpec((C, Cr), lambda b: (0, 0)),
            pl.BlockSpec((Cr, C), lambda b: (0, 0)),
        ],
        out_specs=pl.BlockSpec((4, HW, C), lambda b: (b, 0, 0)),
        compiler_params=pltpu.CompilerParams(
            dimension_semantics=("parallel",),
            vmem_limit_bytes=64 << 20),
    )(x_flat, w1t, w2t)

    return jnp.transpose(out_flat.reshape(B, H, W, C), (0, 3, 1, 2))
